# baseline (speedup 1.0000x reference)
import functools

import jax
import jax.numpy as jnp
from jax import lax
from jax.experimental import pallas as pl
from jax.experimental.pallas import tpu as pltpu
from jax.experimental.pallas import tpu_sc as plsc

F = 26
V = 100000
D = 16
B = 16384

NC = 2
NS = 16
NW = NC * NS

BF = B * F
PER_W = BF // NW
IDX_ROWS = PER_W // 128
RPC = 13
NCH = IDX_ROWS // RPC
CHUNK = RPC * 128


def _body(tab_hbm, idx_hbm, out_hbm, idx_v, rows_v, sem):
    wid = lax.axis_index("s") * NC + lax.axis_index("c")
    row0 = wid * IDX_ROWS
    base = wid * PER_W

    pltpu.sync_copy(idx_hbm.at[pl.ds(row0, IDX_ROWS)], idx_v)

    lanes = lax.iota(jnp.int32, 16)

    def adjust(r):
        p0 = (row0 + r) * 128
        for c in range(8):
            pos = p0 + c * 16 + lanes
            f = lax.rem(pos, F)
            sl = pl.ds(c * 16, 16)
            idx_v[r, sl] = idx_v[r, sl] + f * V

    pl.loop(0, IDX_ROWS)(adjust)

    def chunk(ci):
        copies = []
        for r in range(RPC):
            cp = pltpu.async_copy(
                tab_hbm.at[idx_v.at[ci * RPC + r]],
                rows_v.at[pl.ds(r * 128, 128)],
                sem,
            )
            copies.append(cp)
        for cp in copies:
            cp.wait()
        pltpu.sync_copy(rows_v, out_hbm.at[pl.ds(base + ci * CHUNK, CHUNK)])

    pl.loop(0, NCH)(chunk)


@jax.jit
def _run(tab_flat, idx2d):
    kern = functools.partial(
        pl.kernel,
        mesh=plsc.VectorSubcoreMesh(core_axis_name="c", subcore_axis_name="s"),
        out_type=jax.ShapeDtypeStruct((BF, D), jnp.float32),
        scratch_types=[
            pltpu.VMEM((IDX_ROWS, 128), jnp.int32),
            pltpu.VMEM((CHUNK, D), jnp.float32),
            pltpu.SemaphoreType.DMA,
        ],
        compiler_params=pltpu.CompilerParams(use_tc_tiling_on_sc=False),
    )(_body)
    return kern(tab_flat, idx2d)


def kernel(indices, tables):
    tab_flat = tables.reshape(F * V, D)
    idx2d = indices.reshape(BF // 128, 128).astype(jnp.int32)
    out = _run(tab_flat, idx2d)
    return out.reshape(B, F * D)


# speedup vs baseline: 5.6434x; 5.6434x over previous
import functools

import jax
import jax.numpy as jnp
from jax import lax
from jax.experimental import pallas as pl
from jax.experimental.pallas import tpu as pltpu
from jax.experimental.pallas import tpu_sc as plsc

F = 26
V = 100000
D = 16
B = 16384

NC = 2
NS = 16
NW = NC * NS

ROWS = F * D
PER_W = ROWS // NW
OCH = 4096
NOCH = B // OCH


def _body(tab_hbm, idx_hbm, out_hbm, row_v, idx_v, ob0, ob1, sem0, sem1):
    wid = lax.axis_index("s") * NC + lax.axis_index("c")
    r0 = wid * PER_W

    obufs = (ob0, ob1)
    sems = (sem0, sem1)
    pending = [None, None]

    for j in range(PER_W):
        r = r0 + j
        f = r // D

        if j == 0:
            pltpu.sync_copy(idx_hbm.at[f], idx_v)
        else:
            f_prev = (r - 1) // D

            @pl.when(f != f_prev)
            def _():
                pltpu.sync_copy(idx_hbm.at[f], idx_v)

        pltpu.sync_copy(tab_hbm.at[r], row_v)

        for c in range(NOCH):
            k = c % 2
            if pending[k] is not None:
                pending[k].wait()
            ob = obufs[k]

            def gather(g, c=c, ob=ob):
                iv = idx_v[pl.ds(c * OCH + g * 16, 16)]
                ob[pl.ds(g * 16, 16)] = plsc.load_gather(row_v, [iv])

            pl.loop(0, OCH // 16)(gather)
            pending[k] = pltpu.async_copy(
                ob, out_hbm.at[r, pl.ds(c * OCH, OCH)], sems[k]
            )

    for p in pending:
        if p is not None:
            p.wait()


@jax.jit
def _run(tab_t, idx_t):
    kern = functools.partial(
        pl.kernel,
        mesh=plsc.VectorSubcoreMesh(core_axis_name="c", subcore_axis_name="s"),
        out_type=jax.ShapeDtypeStruct((ROWS, B), jnp.float32),
        scratch_types=[
            pltpu.VMEM((V,), jnp.float32),
            pltpu.VMEM((B,), jnp.int32),
            pltpu.VMEM((OCH,), jnp.float32),
            pltpu.VMEM((OCH,), jnp.float32),
            pltpu.SemaphoreType.DMA,
            pltpu.SemaphoreType.DMA,
        ],
        compiler_params=pltpu.CompilerParams(
            use_tc_tiling_on_sc=True, needs_layout_passes=False
        ),
    )(_body)
    return kern(tab_t, idx_t)


def kernel(indices, tables):
    tab_t = tables.transpose(0, 2, 1).reshape(ROWS, V)
    idx_t = indices.T.astype(jnp.int32)
    out_t = _run(tab_t, idx_t)
    return out_t.T
